# Initial kernel scaffold; baseline (speedup 1.0000x reference)
import functools

import jax
import jax.numpy as jnp
from jax import lax
from jax.experimental import pallas as pl
from jax.experimental.pallas import tpu as pltpu
from jax.experimental.pallas import tpu_sc as plsc

D = 100
CHUNK = 128
NBUF = 4
NC = 2
NS = 16
NW = NC * NS


def _emb_kernel_body(table_hbm, idx_hbm, out_hbm, idx_v, rows_v, gsem, osem):
    n_chunks = idx_hbm.shape[1]
    wid = lax.axis_index("s") * NC + lax.axis_index("c")
    pltpu.sync_copy(idx_hbm.at[wid], idx_v)

    n_groups = n_chunks // NBUF

    def group(g, carry):
        j0 = g * NBUF
        for b in range(NBUF):
            pltpu.async_copy(
                table_hbm.at[idx_v.at[j0 + b]], rows_v.at[b], gsem)
        for b in range(NBUF):
            pltpu.make_async_copy(
                out_hbm.at[wid, 0], rows_v.at[b], gsem).wait()
        for b in range(NBUF):
            pltpu.async_copy(rows_v.at[b], out_hbm.at[wid, j0 + b], osem)
        for b in range(NBUF):
            pltpu.make_async_copy(
                rows_v.at[b], out_hbm.at[wid, 0], osem).wait()
        return carry

    lax.fori_loop(0, n_groups, group, 0)


def kernel(word_embeddings, input_ids, seq_length):
    B0, S = input_ids.shape
    B = B0 * S
    n_chunks = B // (NW * CHUNK)
    idx = input_ids.reshape(-1).astype(jnp.int32).reshape(NW, n_chunks, CHUNK)

    mesh = plsc.VectorSubcoreMesh(core_axis_name="c", subcore_axis_name="s")
    emb = functools.partial(
        pl.kernel,
        mesh=mesh,
        out_type=jax.ShapeDtypeStruct((NW, n_chunks, CHUNK, D), jnp.float32),
        scratch_types=[
            pltpu.VMEM((n_chunks, CHUNK), jnp.int32),
            pltpu.VMEM((NBUF, CHUNK, D), jnp.float32),
            pltpu.SemaphoreType.DMA,
            pltpu.SemaphoreType.DMA,
        ],
    )(_emb_kernel_body)

    out = emb(word_embeddings, idx)
    return out.reshape(B0, S, D)


# speedup vs baseline: 4.8818x; 4.8818x over previous
import functools

import jax
import jax.numpy as jnp
from jax import lax
from jax.experimental import pallas as pl
from jax.experimental.pallas import tpu as pltpu
from jax.experimental.pallas import tpu_sc as plsc

D = 100
DP = 128
CHUNK = 128
NBUF = 4
NC = 2
NS = 16
NW = NC * NS


def _emb_kernel_body(table_hbm, idx_hbm, out_hbm, idx_v, rows_v, gsem, osem):
    n_chunks = idx_hbm.shape[1]
    wid = lax.axis_index("s") * NC + lax.axis_index("c")
    pltpu.sync_copy(idx_hbm.at[wid], idx_v)

    n_groups = n_chunks // NBUF

    def group(g, carry):
        j0 = g * NBUF
        gathers = [
            pltpu.async_copy(
                table_hbm.at[idx_v.at[j0 + b]], rows_v.at[b], gsem)
            for b in range(NBUF)
        ]
        for h in gathers:
            h.wait()
        writes = [
            pltpu.async_copy(rows_v.at[b], out_hbm.at[wid, j0 + b], osem)
            for b in range(NBUF)
        ]
        for h in writes:
            h.wait()
        return carry

    lax.fori_loop(0, n_groups, group, 0)


def kernel(word_embeddings, input_ids, seq_length):
    B0, S = input_ids.shape
    B = B0 * S
    n_chunks = B // (NW * CHUNK)
    idx = input_ids.reshape(-1).astype(jnp.int32).reshape(NW, n_chunks, CHUNK)
    table = jnp.pad(word_embeddings, ((0, 0), (0, DP - D)))

    mesh = plsc.VectorSubcoreMesh(core_axis_name="c", subcore_axis_name="s")
    emb = functools.partial(
        pl.kernel,
        mesh=mesh,
        compiler_params=pltpu.CompilerParams(use_tc_tiling_on_sc=False),
        out_type=jax.ShapeDtypeStruct((NW, n_chunks, CHUNK, DP), jnp.float32),
        scratch_types=[
            pltpu.VMEM((n_chunks, CHUNK), jnp.int32),
            pltpu.VMEM((NBUF, CHUNK, DP), jnp.float32),
            pltpu.SemaphoreType.DMA,
            pltpu.SemaphoreType.DMA,
        ],
    )(_emb_kernel_body)

    out = emb(table, idx)
    return out.reshape(B0, S, DP)[:, :, :D]


# speedup vs baseline: 5.0376x; 1.0319x over previous
import functools

import jax
import jax.numpy as jnp
from jax import lax
from jax.experimental import pallas as pl
from jax.experimental.pallas import tpu as pltpu
from jax.experimental.pallas import tpu_sc as plsc

D = 100
DP = 128
CHUNK = 128
K = 2
NSETS = 2
NC = 2
NS = 16
NW = NC * NS


def _emb_kernel_body(table_hbm, idx_hbm, out_hbm, idx_v, rows_v,
                     gsem, wsem0, wsem1):
    n_chunks = idx_hbm.shape[1]
    wid = lax.axis_index("s") * NC + lax.axis_index("c")
    wsems = (wsem0, wsem1)
    pltpu.sync_copy(idx_hbm.at[wid], idx_v)

    n_groups = n_chunks // K

    def run_group(g, s, first):
        if not first:
            for k in range(K):
                pltpu.make_async_copy(
                    rows_v.at[s * K + k], out_hbm.at[wid, 0], wsems[s]
                ).wait()
        gathers = [
            pltpu.async_copy(
                table_hbm.at[idx_v.at[g * K + k]], rows_v.at[s * K + k], gsem)
            for k in range(K)
        ]
        for h in gathers:
            h.wait()
        for k in range(K):
            pltpu.async_copy(
                rows_v.at[s * K + k], out_hbm.at[wid, g * K + k], wsems[s])

    run_group(0, 0, True)
    run_group(1, 1, True)

    def body(gg, carry):
        run_group(NSETS * gg, 0, False)
        run_group(NSETS * gg + 1, 1, False)
        return carry

    lax.fori_loop(1, n_groups // NSETS, body, 0)

    for s in range(NSETS):
        for k in range(K):
            pltpu.make_async_copy(
                rows_v.at[s * K + k], out_hbm.at[wid, 0], wsems[s]).wait()


def kernel(word_embeddings, input_ids, seq_length):
    B0, S = input_ids.shape
    B = B0 * S
    n_chunks = B // (NW * CHUNK)
    idx = input_ids.reshape(-1).astype(jnp.int32).reshape(NW, n_chunks, CHUNK)
    table = jnp.pad(word_embeddings, ((0, 0), (0, DP - D)))

    mesh = plsc.VectorSubcoreMesh(core_axis_name="c", subcore_axis_name="s")
    emb = functools.partial(
        pl.kernel,
        mesh=mesh,
        compiler_params=pltpu.CompilerParams(use_tc_tiling_on_sc=False),
        out_type=jax.ShapeDtypeStruct((NW, n_chunks, CHUNK, DP), jnp.float32),
        scratch_types=[
            pltpu.VMEM((n_chunks, CHUNK), jnp.int32),
            pltpu.VMEM((NSETS * K, CHUNK, DP), jnp.float32),
            pltpu.SemaphoreType.DMA,
            pltpu.SemaphoreType.DMA,
            pltpu.SemaphoreType.DMA,
        ],
    )(_emb_kernel_body)

    out = emb(table, idx)
    return out.reshape(B0, S, DP)[:, :, :D]
